# baseline (speedup 1.0000x reference)
import functools

import jax
import jax.numpy as jnp
from jax import lax
from jax.experimental import pallas as pl
from jax.experimental.pallas import tpu as pltpu
from jax.experimental.pallas import tpu_sc as plsc

N_NODES = 10000
N_EDGES = 320000
D_FEAT = 128
HIDDEN = 128
N_CLASSES = 64

NC = 2
NS = 16
NW = NC * NS
EPW = N_EDGES // NW
CE = 80
NCHUNK = EPW // CE
G = 25
NG = NCHUNK // G
NB = 3
N_PAD = 10240
STRIPE = N_PAD // NS

_sc_mesh = plsc.VectorSubcoreMesh(
    core_axis_name="c", subcore_axis_name="s", num_cores=NC, num_subcores=NS
)


@functools.partial(
    pl.kernel,
    out_type=jax.ShapeDtypeStruct((NC, N_PAD, D_FEAT), jnp.float32),
    mesh=_sc_mesh,
    scratch_types=[
        pltpu.VMEM((G, CE), jnp.int32),
        pltpu.VMEM((G, CE), jnp.int32),
        pltpu.VMEM((G, CE), jnp.int32),
        pltpu.VMEM((G, CE), jnp.int32),
        pltpu.VMEM((CE, D_FEAT), jnp.float32),
        pltpu.VMEM((CE, D_FEAT), jnp.float32),
        pltpu.VMEM((CE, D_FEAT), jnp.float32),
        pltpu.VMEM_SHARED((N_PAD, D_FEAT), jnp.float32),
        pltpu.SemaphoreType.DMA,
        pltpu.SemaphoreType.DMA,
        pltpu.SemaphoreType.DMA,
        pltpu.SemaphoreType.DMA,
        pltpu.SemaphoreType.DMA,
        pltpu.SemaphoreType.DMA,
        pltpu.SemaphoreType.DMA,
    ],
)
def _sc_aggregate(x_hbm, src_hbm, dst_hbm, out_hbm, src_v0, src_v1, dst_v0,
                  dst_v1, rows_0, rows_1, rows_2, aggr_sh, gs0, gs1, gs2,
                  as0, as1, as2, sem_i):
    c = lax.axis_index("c")
    s = lax.axis_index("s")
    wid = c * NS + s

    rows = (rows_0, rows_1, rows_2)
    gsem = (gs0, gs1, gs2)
    asem = (as0, as1, as2)
    srcg = (src_v0, src_v1)
    dstg = (dst_v0, dst_v1)

    idx_waits = {0: [
        pltpu.async_copy(src_hbm.at[wid * NG], src_v0, sem_i),
        pltpu.async_copy(dst_hbm.at[wid * NG], dst_v0, sem_i),
    ]}

    zvec = jnp.zeros((16,), jnp.float32)

    def _zb_body(r, carry):
        for j in range(D_FEAT // 16):
            rows_0[r, pl.ds(j * 16, 16)] = zvec
        return carry

    lax.fori_loop(0, CE, _zb_body, 0)
    row0 = s * STRIPE
    zw = [
        pltpu.async_copy(rows_0, aggr_sh.at[pl.ds(row0 + k * CE, CE)], gs1)
        for k in range(STRIPE // CE)
    ]
    for w in zw:
        w.wait()
    for w in idx_waits[0]:
        w.wait()
    plsc.subcore_barrier()

    def _gather(n):
        b = n % NB
        g, j = divmod(n, G)
        pltpu.async_copy(x_hbm.at[srcg[g % 2].at[j]], rows[b], gsem[b])

    def _wait_gather(n):
        b = n % NB
        pltpu.make_async_copy(x_hbm.at[src_v0.at[0]], rows[b], gsem[b]).wait()

    def _add(n):
        b = n % NB
        g, j = divmod(n, G)
        pltpu.async_copy(rows[b], aggr_sh.at[dstg[g % 2].at[j]], asem[b],
                         add=True)

    def _wait_add(n):
        b = n % NB
        pltpu.make_async_copy(rows[b], aggr_sh.at[dst_v0.at[0]],
                              asem[b]).wait()

    _gather(0)
    _gather(1)
    for n in range(NCHUNK):
        g, j = divmod(n, G)
        if j == 0 and g + 1 < NG:
            idx_waits[g + 1] = [
                pltpu.async_copy(src_hbm.at[wid * NG + g + 1],
                                 srcg[(g + 1) % 2], sem_i),
                pltpu.async_copy(dst_hbm.at[wid * NG + g + 1],
                                 dstg[(g + 1) % 2], sem_i),
            ]
        _wait_gather(n)
        _add(n)
        m = n + 2
        if m < NCHUNK:
            if m % G == 0:
                for w in idx_waits[m // G]:
                    w.wait()
            if m >= NB:
                _wait_add(m - NB)
            _gather(m)
    for n in range(NCHUNK - NB, NCHUNK):
        _wait_add(n)
    plsc.subcore_barrier()

    pltpu.sync_copy(
        aggr_sh.at[pl.ds(row0, STRIPE)], out_hbm.at[c, pl.ds(row0, STRIPE)]
    )


BLK = 2000


def _bf(v):
    return v.astype(jnp.bfloat16)


def _mlp1_body(x_ref, a0_ref, a1_ref, wa_ref, ba_ref, wb_ref, bb_ref, o_ref):
    t = x_ref[...] + a0_ref[0] + a1_ref[0]
    h = jnp.dot(_bf(t), _bf(wa_ref[...]),
                preferred_element_type=jnp.float32) + ba_ref[...]
    h = jnp.maximum(h, 0.0)
    h = jnp.dot(_bf(h), _bf(wb_ref[...]),
                preferred_element_type=jnp.float32) + bb_ref[...]
    o_ref[...] = jnp.maximum(h, 0.0)


def _mlp2_body(h_ref, a0_ref, a1_ref, wa_ref, ba_ref, wb_ref, bb_ref,
               wfc_ref, bfc_ref, o_ref):
    t = h_ref[...] + a0_ref[0] + a1_ref[0]
    u = jnp.dot(_bf(t), _bf(wa_ref[...]),
                preferred_element_type=jnp.float32) + ba_ref[...]
    u = jnp.maximum(u, 0.0)
    u = jnp.dot(_bf(u), _bf(wb_ref[...]),
                preferred_element_type=jnp.float32) + bb_ref[...]
    o_ref[...] = (
        jnp.dot(_bf(u), _bf(wfc_ref[...]),
                preferred_element_type=jnp.float32) + bfc_ref[...]
    )


def _row_spec(d):
    return pl.BlockSpec((BLK, d), lambda i: (i, 0))


def _part_spec(p, d):
    return pl.BlockSpec((1, BLK, d), lambda i, p=p: (p, i, 0))


def _full_spec(r, d):
    return pl.BlockSpec((r, d), lambda i: (0, 0))


_mlp1 = pl.pallas_call(
    _mlp1_body,
    grid=(N_NODES // BLK,),
    in_specs=[
        _row_spec(D_FEAT), _part_spec(0, D_FEAT), _part_spec(1, D_FEAT),
        _full_spec(D_FEAT, HIDDEN), _full_spec(1, HIDDEN),
        _full_spec(HIDDEN, HIDDEN), _full_spec(1, HIDDEN),
    ],
    out_specs=_row_spec(HIDDEN),
    out_shape=jax.ShapeDtypeStruct((N_NODES, HIDDEN), jnp.float32),
)

_mlp2 = pl.pallas_call(
    _mlp2_body,
    grid=(N_NODES // BLK,),
    in_specs=[
        _row_spec(HIDDEN), _part_spec(0, HIDDEN), _part_spec(1, HIDDEN),
        _full_spec(HIDDEN, HIDDEN), _full_spec(1, HIDDEN),
        _full_spec(HIDDEN, HIDDEN), _full_spec(1, HIDDEN),
        _full_spec(HIDDEN, N_CLASSES), _full_spec(1, N_CLASSES),
    ],
    out_specs=_row_spec(N_CLASSES),
    out_shape=jax.ShapeDtypeStruct((N_NODES, N_CLASSES), jnp.float32),
)


def kernel(x, edge_index, W1a, b1a, W1b, b1b, W2a, b2a, W2b, b2b, Wfc, bfc):
    src = edge_index[0].astype(jnp.int32).reshape(NW * NG, G, CE)
    dst = edge_index[1].astype(jnp.int32).reshape(NW * NG, G, CE)

    a = _sc_aggregate(x, src, dst)
    h1 = _mlp1(x, a, a,
               W1a, b1a.reshape(1, -1), W1b, b1b.reshape(1, -1))
    b = _sc_aggregate(h1, src, dst)
    out = _mlp2(h1, b, b,
                W2a, b2a.reshape(1, -1), W2b, b2b.reshape(1, -1),
                Wfc, bfc.reshape(1, -1))
    return out


# speedup vs baseline: 1.0145x; 1.0145x over previous
import functools

import jax
import jax.numpy as jnp
from jax import lax
from jax.experimental import pallas as pl
from jax.experimental.pallas import tpu as pltpu
from jax.experimental.pallas import tpu_sc as plsc

N_NODES = 10000
N_EDGES = 320000
D_FEAT = 128
HIDDEN = 128
N_CLASSES = 64

NC = 2
NS = 16
NW = NC * NS
EPW = N_EDGES // NW
CE = 80
NCHUNK = EPW // CE
G = 25
NG = NCHUNK // G
NB = 3
N_PAD = 10240
STRIPE = N_PAD // NS

_sc_mesh = plsc.VectorSubcoreMesh(
    core_axis_name="c", subcore_axis_name="s", num_cores=NC, num_subcores=NS
)


@functools.partial(
    pl.kernel,
    out_type=jax.ShapeDtypeStruct((NC, N_PAD, D_FEAT), jnp.float32),
    mesh=_sc_mesh,
    scratch_types=[
        pltpu.VMEM((G, CE), jnp.int32),
        pltpu.VMEM((G, CE), jnp.int32),
        pltpu.VMEM((G, CE), jnp.int32),
        pltpu.VMEM((G, CE), jnp.int32),
        pltpu.VMEM((CE, D_FEAT), jnp.float32),
        pltpu.VMEM((CE, D_FEAT), jnp.float32),
        pltpu.VMEM((CE, D_FEAT), jnp.float32),
        pltpu.VMEM_SHARED((N_PAD, D_FEAT), jnp.float32),
        pltpu.SemaphoreType.DMA,
        pltpu.SemaphoreType.DMA,
        pltpu.SemaphoreType.DMA,
        pltpu.SemaphoreType.DMA,
        pltpu.SemaphoreType.DMA,
        pltpu.SemaphoreType.DMA,
        pltpu.SemaphoreType.DMA,
    ],
)
def _sc_aggregate(x_hbm, src_hbm, dst_hbm, out_hbm, src_v0, src_v1, dst_v0,
                  dst_v1, rows_0, rows_1, rows_2, aggr_sh, gs0, gs1, gs2,
                  as0, as1, as2, sem_i):
    c = lax.axis_index("c")
    s = lax.axis_index("s")
    wid = c * NS + s

    rows = (rows_0, rows_1, rows_2)
    gsem = (gs0, gs1, gs2)
    asem = (as0, as1, as2)
    srcg = (src_v0, src_v1)
    dstg = (dst_v0, dst_v1)

    idx_waits = {0: [
        pltpu.async_copy(src_hbm.at[wid * NG], src_v0, sem_i),
        pltpu.async_copy(dst_hbm.at[wid * NG], dst_v0, sem_i),
    ]}

    zvec = jnp.zeros((16,), jnp.float32)

    def _zb_body(r, carry):
        for j in range(D_FEAT // 16):
            rows_0[r, pl.ds(j * 16, 16)] = zvec
        return carry

    lax.fori_loop(0, CE, _zb_body, 0)
    row0 = s * STRIPE
    zw = [
        pltpu.async_copy(rows_0, aggr_sh.at[pl.ds(row0 + k * CE, CE)], gs1)
        for k in range(STRIPE // CE)
    ]
    for w in zw:
        w.wait()
    for w in idx_waits[0]:
        w.wait()
    plsc.subcore_barrier()

    def _gather(n):
        b = n % NB
        g, j = divmod(n, G)
        pltpu.async_copy(x_hbm.at[srcg[g % 2].at[j]], rows[b], gsem[b])

    def _wait_gather(n):
        b = n % NB
        pltpu.make_async_copy(x_hbm.at[src_v0.at[0]], rows[b], gsem[b]).wait()

    def _add(n):
        b = n % NB
        g, j = divmod(n, G)
        pltpu.async_copy(rows[b], aggr_sh.at[dstg[g % 2].at[j]], asem[b],
                         add=True)

    def _wait_add(n):
        b = n % NB
        pltpu.make_async_copy(rows[b], aggr_sh.at[dst_v0.at[0]],
                              asem[b]).wait()

    _gather(0)
    _gather(1)
    for n in range(NCHUNK):
        g, j = divmod(n, G)
        if j == 0 and g + 1 < NG:
            idx_waits[g + 1] = [
                pltpu.async_copy(src_hbm.at[wid * NG + g + 1],
                                 srcg[(g + 1) % 2], sem_i),
                pltpu.async_copy(dst_hbm.at[wid * NG + g + 1],
                                 dstg[(g + 1) % 2], sem_i),
            ]
        _wait_gather(n)
        _add(n)
        m = n + 2
        if m < NCHUNK:
            if m % G == 0:
                for w in idx_waits[m // G]:
                    w.wait()
            if m >= NB:
                _wait_add(m - NB)
            _gather(m)
    for n in range(NCHUNK - NB, NCHUNK):
        _wait_add(n)
    plsc.subcore_barrier()

    pltpu.sync_copy(
        aggr_sh.at[pl.ds(row0, STRIPE)], out_hbm.at[c, pl.ds(row0, STRIPE)]
    )


BLK = 5000


def _bf(v):
    return v.astype(jnp.bfloat16)


def _mlp1_body(x_ref, a0_ref, a1_ref, wa_ref, ba_ref, wb_ref, bb_ref, o_ref):
    t = x_ref[...] + a0_ref[0] + a1_ref[0]
    h = jnp.dot(_bf(t), _bf(wa_ref[...]),
                preferred_element_type=jnp.float32) + ba_ref[...]
    h = jnp.maximum(h, 0.0)
    h = jnp.dot(_bf(h), _bf(wb_ref[...]),
                preferred_element_type=jnp.float32) + bb_ref[...]
    o_ref[...] = jnp.maximum(h, 0.0)


def _mlp2_body(h_ref, a0_ref, a1_ref, wa_ref, ba_ref, wb_ref, bb_ref,
               wfc_ref, bfc_ref, o_ref):
    t = h_ref[...] + a0_ref[0] + a1_ref[0]
    u = jnp.dot(_bf(t), _bf(wa_ref[...]),
                preferred_element_type=jnp.float32) + ba_ref[...]
    u = jnp.maximum(u, 0.0)
    u = jnp.dot(_bf(u), _bf(wb_ref[...]),
                preferred_element_type=jnp.float32) + bb_ref[...]
    o_ref[...] = (
        jnp.dot(_bf(u), _bf(wfc_ref[...]),
                preferred_element_type=jnp.float32) + bfc_ref[...]
    )


def _row_spec(d):
    return pl.BlockSpec((BLK, d), lambda i: (i, 0))


def _part_spec(p, d):
    return pl.BlockSpec((1, BLK, d), lambda i, p=p: (p, i, 0))


def _full_spec(r, d):
    return pl.BlockSpec((r, d), lambda i: (0, 0))


_mlp1 = pl.pallas_call(
    _mlp1_body,
    grid=(N_NODES // BLK,),
    in_specs=[
        _row_spec(D_FEAT), _part_spec(0, D_FEAT), _part_spec(1, D_FEAT),
        _full_spec(D_FEAT, HIDDEN), _full_spec(1, HIDDEN),
        _full_spec(HIDDEN, HIDDEN), _full_spec(1, HIDDEN),
    ],
    out_specs=_row_spec(HIDDEN),
    out_shape=jax.ShapeDtypeStruct((N_NODES, HIDDEN), jnp.float32),
)

_mlp2 = pl.pallas_call(
    _mlp2_body,
    grid=(N_NODES // BLK,),
    in_specs=[
        _row_spec(HIDDEN), _part_spec(0, HIDDEN), _part_spec(1, HIDDEN),
        _full_spec(HIDDEN, HIDDEN), _full_spec(1, HIDDEN),
        _full_spec(HIDDEN, HIDDEN), _full_spec(1, HIDDEN),
        _full_spec(HIDDEN, N_CLASSES), _full_spec(1, N_CLASSES),
    ],
    out_specs=_row_spec(N_CLASSES),
    out_shape=jax.ShapeDtypeStruct((N_NODES, N_CLASSES), jnp.float32),
)


def kernel(x, edge_index, W1a, b1a, W1b, b1b, W2a, b2a, W2b, b2b, Wfc, bfc):
    src = edge_index[0].astype(jnp.int32).reshape(NW * NG, G, CE)
    dst = edge_index[1].astype(jnp.int32).reshape(NW * NG, G, CE)

    a = _sc_aggregate(x, src, dst)
    h1 = _mlp1(x, a, a,
               W1a, b1a.reshape(1, -1), W1b, b1b.reshape(1, -1))
    b = _sc_aggregate(h1, src, dst)
    out = _mlp2(h1, b, b,
                W2a, b2a.reshape(1, -1), W2b, b2b.reshape(1, -1),
                Wfc, bfc.reshape(1, -1))
    return out
